# baseline (speedup 1.0000x reference)
import functools

import jax
import jax.numpy as jnp
import numpy as np
from jax.experimental import pallas as pl
from jax.experimental.pallas import tpu as pltpu

B, S, H = 2, 2048, 1024
NHS, HD = 4, 64
HS = NHS * HD
E, K = 8, 2
INTER = 768
EPS = 1e-05

CHUNK = 256
NCHUNK = S // CHUNK


def _rms(x):
    var = jnp.mean(x * x, axis=-1, keepdims=True)
    return x * jax.lax.rsqrt(var + EPS)


def _add_kernel(a_ref, b_ref, o_ref):
    o_ref[...] = a_ref[...] + b_ref[...]


def _make_hs(hidden, inj):
    blk = pl.BlockSpec((1, 512, H), lambda b, c: (b, c, 0))
    return pl.pallas_call(
        _add_kernel,
        grid=(B, S // 512),
        in_specs=[blk, blk],
        out_specs=blk,
        out_shape=jax.ShapeDtypeStruct((B, S, H), jnp.float32),
    )(hidden, inj)


def _router_kernel(hs0_ref, wg_ref, topi_ref, wv_ref, aux_ref):
    x = hs0_ref[...]
    logits = jax.lax.dot_general(
        x, wg_ref[...], (((1,), (0,)), ((), ())),
        preferred_element_type=jnp.float32)
    m = jnp.max(logits, axis=1, keepdims=True)
    ex = jnp.exp(logits - m)
    p = ex / jnp.sum(ex, axis=1, keepdims=True)

    iota = jax.lax.broadcasted_iota(jnp.int32, (B, E), 1)
    v1 = jnp.max(p, axis=1, keepdims=True)
    i1 = jnp.min(jnp.where(p == v1, iota, E), axis=1, keepdims=True)
    mask1 = iota == i1
    p2 = jnp.where(mask1, -1.0, p)
    v2 = jnp.max(p2, axis=1, keepdims=True)
    i2 = jnp.min(jnp.where(p2 == v2, iota, E), axis=1, keepdims=True)
    mask2 = iota == i2

    denom = jnp.maximum(v1 + v2, 1e-08)
    wv_ref[...] = jnp.concatenate([v1 / denom, v2 / denom], axis=1)
    topi_ref[...] = jnp.concatenate([i1, i2], axis=1)

    importance = jnp.sum(p, axis=0, keepdims=True) / B
    sel = (mask1 | mask2).astype(jnp.float32)
    load = jnp.sum(sel, axis=0, keepdims=True) / (B * K)
    aux_ref[...] = jnp.sum(E * importance * load, axis=1, keepdims=True)


def _route(hs0, wg):
    return pl.pallas_call(
        _router_kernel,
        in_specs=[pl.BlockSpec(memory_space=pltpu.VMEM),
                  pl.BlockSpec(memory_space=pltpu.VMEM)],
        out_specs=[pl.BlockSpec(memory_space=pltpu.VMEM)] * 3,
        out_shape=[
            jax.ShapeDtypeStruct((B, K), jnp.int32),
            jax.ShapeDtypeStruct((B, K), jnp.float32),
            jax.ShapeDtypeStruct((1, 1), jnp.float32),
        ],
    )(hs0, wg)


def _moe_kernel(topi_ref, wv_ref,
                hs_ref, cos_ref, sin_ref,
                wd_ref, wqkv_ref, wo_ref, wgu_ref, wdn_ref, wu_ref,
                out_ref,
                h_s, q_s, k_s, v_s):
    b = pl.program_id(0)
    kk = pl.program_id(1)
    wgt = wv_ref[b, kk]

    bf = jnp.bfloat16
    wd = wd_ref[0].astype(bf)
    wqkv = wqkv_ref[0].astype(bf)
    scale = np.float32(1.0 / np.sqrt(HD))

    def rope_head(xh, cosc, sinc):
        x1 = xh[:, :HD // 2]
        x2 = xh[:, HD // 2:]
        rot = jnp.concatenate([-x2, x1], axis=1)
        return xh * cosc + rot * sinc

    def phase1(c):
        rows = pl.ds(c * CHUNK, CHUNK)
        x = hs_ref[0, rows, :].astype(bf)
        hc = jnp.dot(x, wd, preferred_element_type=jnp.float32)
        h_s[rows, :] = hc
        qkv = jnp.dot(hc.astype(bf), wqkv,
                      preferred_element_type=jnp.float32)
        cosc = cos_ref[rows, :]
        sinc = sin_ref[rows, :]
        for hh in range(NHS):
            qh = qkv[:, hh * HD:(hh + 1) * HD]
            kh = qkv[:, HS + hh * HD:HS + (hh + 1) * HD]
            vh = qkv[:, 2 * HS + hh * HD:2 * HS + (hh + 1) * HD]
            q_s[hh, rows, :] = (rope_head(qh, cosc, sinc) * scale).astype(bf)
            k_s[hh, rows, :] = rope_head(kh, cosc, sinc).astype(bf)
            v_s[hh, rows, :] = vh.astype(bf)

    for c in range(NCHUNK):
        phase1(c)

    wo = wo_ref[0].astype(bf)
    wgu = wgu_ref[0].astype(bf)
    wdn = wdn_ref[0].astype(bf)
    wu = wu_ref[0].astype(bf)

    def phase2(c):
        rows = pl.ds(c * CHUNK, CHUNK)
        hc = h_s[rows, :]
        o_heads = []
        for hh in range(NHS):
            qh = q_s[hh, rows, :]
            kh = k_s[hh]
            vh = v_s[hh]
            scores = jax.lax.dot_general(
                qh, kh, (((1,), (1,)), ((), ())),
                preferred_element_type=jnp.float32)
            mx = jnp.max(scores, axis=1, keepdims=True)
            ee = jnp.exp(scores - mx)
            rs = 1.0 / jnp.sum(ee, axis=1, keepdims=True)
            ov = jnp.dot(ee.astype(bf), vh,
                         preferred_element_type=jnp.float32)
            o_heads.append(ov * rs)
        o = jnp.concatenate(o_heads, axis=1).astype(bf)
        o = jnp.dot(o, wo, preferred_element_type=jnp.float32)
        t = _rms(hc + o)
        gu = jnp.dot(t.astype(bf), wgu,
                     preferred_element_type=jnp.float32)
        g = gu[:, :INTER]
        u = gu[:, INTER:]
        mm = jnp.dot((jax.nn.silu(g) * u).astype(bf), wdn,
                     preferred_element_type=jnp.float32)
        t2 = _rms(t + mm)
        oe = jnp.dot(t2.astype(bf), wu, preferred_element_type=jnp.float32)

        @pl.when(kk == 0)
        def _():
            out_ref[0, rows, :] = wgt * oe

        @pl.when(kk == K - 1)
        def _():
            x = hs_ref[0, rows, :] + out_ref[0, rows, :] + wgt * oe
            var = jnp.mean(x * x, axis=-1, keepdims=True)
            out_ref[0, rows, :] = x * jax.lax.rsqrt(var + EPS)

    for c in range(NCHUNK):
        phase2(c)


def _moe(topi, wv, hs, cos, sin, Wd, Wqkv, Wo, Wgu, Wdn, Wu):
    grid_spec = pltpu.PrefetchScalarGridSpec(
        num_scalar_prefetch=2,
        grid=(B, K),
        in_specs=[
            pl.BlockSpec((1, S, H), lambda b, k, ti, wv_: (b, 0, 0)),
            pl.BlockSpec((S, HD), lambda b, k, ti, wv_: (0, 0)),
            pl.BlockSpec((S, HD), lambda b, k, ti, wv_: (0, 0)),
            pl.BlockSpec((1, H, HS), lambda b, k, ti, wv_: (ti[b, k], 0, 0)),
            pl.BlockSpec((1, HS, 3 * HS), lambda b, k, ti, wv_: (ti[b, k], 0, 0)),
            pl.BlockSpec((1, HS, HS), lambda b, k, ti, wv_: (ti[b, k], 0, 0)),
            pl.BlockSpec((1, HS, 2 * INTER), lambda b, k, ti, wv_: (ti[b, k], 0, 0)),
            pl.BlockSpec((1, INTER, HS), lambda b, k, ti, wv_: (ti[b, k], 0, 0)),
            pl.BlockSpec((1, HS, H), lambda b, k, ti, wv_: (ti[b, k], 0, 0)),
        ],
        out_specs=pl.BlockSpec((1, S, H), lambda b, k, ti, wv_: (b, 0, 0)),
        scratch_shapes=[
            pltpu.VMEM((S, HS), jnp.float32),
            pltpu.VMEM((NHS, S, HD), jnp.bfloat16),
            pltpu.VMEM((NHS, S, HD), jnp.bfloat16),
            pltpu.VMEM((NHS, S, HD), jnp.bfloat16),
        ],
    )
    return pl.pallas_call(
        _moe_kernel,
        grid_spec=grid_spec,
        out_shape=jax.ShapeDtypeStruct((B, S, H), jnp.float32),
        compiler_params=pltpu.CompilerParams(
            vmem_limit_bytes=64 * 1024 * 1024,
            dimension_semantics=("parallel", "arbitrary"),
        ),
    )(topi, wv, hs, cos, sin, Wd, Wqkv, Wo, Wgu, Wdn, Wu)


def _final_kernel(hs_ref, mix_ref, o_ref):
    x = hs_ref[...] + mix_ref[...]
    var = jnp.mean(x * x, axis=-1, keepdims=True)
    o_ref[...] = x * jax.lax.rsqrt(var + EPS)


def _finalize(hs, mixed):
    blk = pl.BlockSpec((1, 512, H), lambda b, c: (b, c, 0))
    return pl.pallas_call(
        _final_kernel,
        grid=(B, S // 512),
        in_specs=[blk, blk],
        out_specs=blk,
        out_shape=jax.ShapeDtypeStruct((B, S, H), jnp.float32),
    )(hs, mixed)


def kernel(hidden_states, input_injection, cos, sin, Wg, Wd, Wu, Wqkv, Wo, Wgu, Wdn):
    hs = _make_hs(hidden_states, input_injection)
    topi, wv, aux = _route(hs[:, 0, :], Wg)
    out = _moe(topi, wv, hs, cos, sin, Wd, Wqkv, Wo, Wgu, Wdn, Wu)
    return out, aux.reshape(())


# speedup vs baseline: 1.0602x; 1.0602x over previous
import dataclasses
import functools

import jax
import jax.numpy as jnp
import numpy as np
from jax.experimental import pallas as pl
from jax.experimental.pallas import tpu as pltpu
from jax.experimental.pallas import tpu_sc as plsc

B, S, H = 2, 2048, 1024
NHS, HD = 4, 64
HS = NHS * HD
E, K = 8, 2
INTER = 768
EPS = 1e-05

CHUNK = 256
NCHUNK = S // CHUNK


def _rms(x):
    var = jnp.mean(x * x, axis=-1, keepdims=True)
    return x * jax.lax.rsqrt(var + EPS)


def _add_kernel(a_ref, b_ref, o_ref):
    o_ref[...] = a_ref[...] + b_ref[...]


def _make_hs(hidden, inj):
    blk = pl.BlockSpec((1, 512, H), lambda b, c: (b, c, 0))
    return pl.pallas_call(
        _add_kernel,
        grid=(B, S // 512),
        in_specs=[blk, blk],
        out_specs=blk,
        out_shape=jax.ShapeDtypeStruct((B, S, H), jnp.float32),
    )(hidden, inj)


def _route(h0, i0, wgT):
    mesh = plsc.VectorSubcoreMesh(core_axis_name="c", subcore_axis_name="s")
    out_type = [
        jax.ShapeDtypeStruct((1, 16), jnp.int32),
        jax.ShapeDtypeStruct((1, 16), jnp.float32),
        jax.ShapeDtypeStruct((1, 16), jnp.float32),
    ]

    cp = pltpu.CompilerParams()
    if "needs_layout_passes" in pltpu.CompilerParams.__dataclass_fields__:
        cp = dataclasses.replace(cp, needs_layout_passes=False)

    @functools.partial(
        pl.kernel, out_type=out_type, mesh=mesh, compiler_params=cp,
        scratch_types=[
            pltpu.VMEM((B, H), jnp.float32),
            pltpu.VMEM((B, H), jnp.float32),
            pltpu.VMEM((E, H), jnp.float32),
            pltpu.VMEM((1, 16), jnp.int32),
            pltpu.VMEM((1, 16), jnp.float32),
            pltpu.VMEM((1, 16), jnp.float32),
            pltpu.SemaphoreType.DMA,
        ])
    def krn(h0_hbm, i0_hbm, wg_hbm, topi_hbm, wv_hbm, aux_hbm,
            h0_v, i0_v, wg_v, ti_v, wv_v, aux_v, sem):
        cid = jax.lax.axis_index("c")
        sid = jax.lax.axis_index("s")

        @pl.when(jnp.logical_and(cid == 0, sid == 0))
        def _():
            pltpu.async_copy(h0_hbm, h0_v, sem).wait()
            pltpu.async_copy(i0_hbm, i0_v, sem).wait()
            pltpu.async_copy(wg_hbm, wg_v, sem).wait()
            lanes = jax.lax.iota(jnp.int32, 16)
            imp = jnp.zeros((16,), jnp.float32)
            ld = jnp.zeros((16,), jnp.float32)
            ti_vec = jnp.zeros((16,), jnp.int32)
            wv_vec = jnp.zeros((16,), jnp.float32)
            for b_ in range(B):
                lg = jnp.full((16,), -3e38, jnp.float32)
                for e_ in range(E):
                    def body(j, acc):
                        off = pl.multiple_of(j * 16, 16)
                        x = h0_v[b_, pl.ds(off, 16)] + i0_v[b_, pl.ds(off, 16)]
                        return acc + x * wg_v[e_, pl.ds(off, 16)]
                    acc = jax.lax.fori_loop(0, H // 16, body,
                                            jnp.zeros((16,), jnp.float32))
                    lg = jnp.where(lanes == e_, jnp.sum(acc), lg)
                m = jnp.max(lg)
                p = jnp.where(lanes < E, jnp.exp(lg - m), 0.0)
                p = p / jnp.broadcast_to(jnp.sum(p), (16,))
                v1 = jnp.max(p)
                i1 = jnp.min(jnp.where(p == v1, lanes, 16))
                p2 = jnp.where(lanes == i1, -1.0, p)
                v2 = jnp.max(p2)
                i2 = jnp.min(jnp.where(p2 == v2, lanes, 16))
                den = jnp.broadcast_to(jnp.maximum(v1 + v2, 1e-08), (16,))
                num = (jnp.where(lanes == K * b_, v1, 0.0)
                       + jnp.where(lanes == K * b_ + 1, v2, 0.0))
                ti_vec = jnp.where(lanes == K * b_, i1, ti_vec)
                ti_vec = jnp.where(lanes == K * b_ + 1, i2, ti_vec)
                wv_vec = wv_vec + num / den
                imp = imp + p
                ld = ld + jnp.where(
                    jnp.logical_or(lanes == i1, lanes == i2), 1.0, 0.0)
            aux = jnp.sum(imp * ld) * (float(E) / (B * B * K))
            ti_v[0, pl.ds(0, 16)] = ti_vec
            wv_v[0, pl.ds(0, 16)] = wv_vec
            aux_v[0, pl.ds(0, 16)] = jnp.where(lanes == 0, aux, 0.0)
            pltpu.async_copy(ti_v, topi_hbm, sem).wait()
            pltpu.async_copy(wv_v, wv_hbm, sem).wait()
            pltpu.async_copy(aux_v, aux_hbm, sem).wait()

    ti_raw, wv_raw, aux_raw = krn(h0, i0, wgT)
    topi = ti_raw[0, :B * K].reshape(B, K)
    wv = wv_raw[0, :B * K].reshape(B, K)
    return topi, wv, aux_raw[0:1, 0:1]


def _moe_kernel(topi_ref, wv_ref,
                hs_ref, cos_ref, sin_ref,
                wd_ref, wqkv_ref, wo_ref, wgu_ref, wdn_ref, wu_ref,
                out_ref,
                h_s, q_s, k_s, v_s):
    b = pl.program_id(0)
    kk = pl.program_id(1)
    wgt = wv_ref[b, kk]

    bf = jnp.bfloat16
    wd = wd_ref[0].astype(bf)
    wqkv = wqkv_ref[0].astype(bf)
    scale = np.float32(1.0 / np.sqrt(HD))

    def rope_head(xh, cosc, sinc):
        x1 = xh[:, :HD // 2]
        x2 = xh[:, HD // 2:]
        rot = jnp.concatenate([-x2, x1], axis=1)
        return xh * cosc + rot * sinc

    def phase1(c, carry):
        rows = pl.ds(c * CHUNK, CHUNK)
        x = hs_ref[0, rows, :].astype(bf)
        hc = jnp.dot(x, wd, preferred_element_type=jnp.float32)
        h_s[rows, :] = hc
        qkv = jnp.dot(hc.astype(bf), wqkv,
                      preferred_element_type=jnp.float32)
        cosc = cos_ref[rows, :]
        sinc = sin_ref[rows, :]
        for hh in range(NHS):
            qh = qkv[:, hh * HD:(hh + 1) * HD]
            kh = qkv[:, HS + hh * HD:HS + (hh + 1) * HD]
            vh = qkv[:, 2 * HS + hh * HD:2 * HS + (hh + 1) * HD]
            q_s[hh, rows, :] = (rope_head(qh, cosc, sinc) * scale).astype(bf)
            k_s[hh, rows, :] = rope_head(kh, cosc, sinc).astype(bf)
            v_s[hh, rows, :] = vh.astype(bf)
        return carry

    jax.lax.fori_loop(0, NCHUNK, phase1, 0)

    wo = wo_ref[0].astype(bf)
    wgu = wgu_ref[0].astype(bf)
    wdn = wdn_ref[0].astype(bf)
    wu = wu_ref[0].astype(bf)

    def phase2(c, carry):
        rows = pl.ds(c * CHUNK, CHUNK)
        hc = h_s[rows, :]
        o_heads = []
        for hh in range(NHS):
            qh = q_s[hh, rows, :]
            kh = k_s[hh]
            vh = v_s[hh]
            scores = jax.lax.dot_general(
                qh, kh, (((1,), (1,)), ((), ())),
                preferred_element_type=jnp.float32)
            mx = jnp.max(scores, axis=1, keepdims=True)
            ee = jnp.exp(scores - mx)
            rs = 1.0 / jnp.sum(ee, axis=1, keepdims=True)
            ov = jnp.dot(ee.astype(bf), vh,
                         preferred_element_type=jnp.float32)
            o_heads.append(ov * rs)
        o = jnp.concatenate(o_heads, axis=1).astype(bf)
        o = jnp.dot(o, wo, preferred_element_type=jnp.float32)
        t = _rms(hc + o)
        gu = jnp.dot(t.astype(bf), wgu,
                     preferred_element_type=jnp.float32)
        g = gu[:, :INTER]
        u = gu[:, INTER:]
        mm = jnp.dot((jax.nn.silu(g) * u).astype(bf), wdn,
                     preferred_element_type=jnp.float32)
        t2 = _rms(t + mm)
        oe = jnp.dot(t2.astype(bf), wu, preferred_element_type=jnp.float32)

        @pl.when(kk == 0)
        def _():
            out_ref[0, rows, :] = wgt * oe

        @pl.when(kk == K - 1)
        def _():
            x = hs_ref[0, rows, :] + out_ref[0, rows, :] + wgt * oe
            var = jnp.mean(x * x, axis=-1, keepdims=True)
            out_ref[0, rows, :] = x * jax.lax.rsqrt(var + EPS)

        return carry

    jax.lax.fori_loop(0, NCHUNK, phase2, 0)


def _moe(topi, wv, hs, cos, sin, Wd, Wqkv, Wo, Wgu, Wdn, Wu):
    grid_spec = pltpu.PrefetchScalarGridSpec(
        num_scalar_prefetch=2,
        grid=(B, K),
        in_specs=[
            pl.BlockSpec((1, S, H), lambda b, k, ti, wv_: (b, 0, 0)),
            pl.BlockSpec((S, HD), lambda b, k, ti, wv_: (0, 0)),
            pl.BlockSpec((S, HD), lambda b, k, ti, wv_: (0, 0)),
            pl.BlockSpec((1, H, HS), lambda b, k, ti, wv_: (ti[b, k], 0, 0)),
            pl.BlockSpec((1, HS, 3 * HS), lambda b, k, ti, wv_: (ti[b, k], 0, 0)),
            pl.BlockSpec((1, HS, HS), lambda b, k, ti, wv_: (ti[b, k], 0, 0)),
            pl.BlockSpec((1, HS, 2 * INTER), lambda b, k, ti, wv_: (ti[b, k], 0, 0)),
            pl.BlockSpec((1, INTER, HS), lambda b, k, ti, wv_: (ti[b, k], 0, 0)),
            pl.BlockSpec((1, HS, H), lambda b, k, ti, wv_: (ti[b, k], 0, 0)),
        ],
        out_specs=pl.BlockSpec((1, S, H), lambda b, k, ti, wv_: (b, 0, 0)),
        scratch_shapes=[
            pltpu.VMEM((S, HS), jnp.float32),
            pltpu.VMEM((NHS, S, HD), jnp.bfloat16),
            pltpu.VMEM((NHS, S, HD), jnp.bfloat16),
            pltpu.VMEM((NHS, S, HD), jnp.bfloat16),
        ],
    )
    return pl.pallas_call(
        _moe_kernel,
        grid_spec=grid_spec,
        out_shape=jax.ShapeDtypeStruct((B, S, H), jnp.float32),
        compiler_params=pltpu.CompilerParams(
            vmem_limit_bytes=64 * 1024 * 1024,
            dimension_semantics=("parallel", "arbitrary"),
        ),
    )(topi, wv, hs, cos, sin, Wd, Wqkv, Wo, Wgu, Wdn, Wu)


def _final_kernel(hs_ref, mix_ref, o_ref):
    x = hs_ref[...] + mix_ref[...]
    var = jnp.mean(x * x, axis=-1, keepdims=True)
    o_ref[...] = x * jax.lax.rsqrt(var + EPS)


def _finalize(hs, mixed):
    blk = pl.BlockSpec((1, 512, H), lambda b, c: (b, c, 0))
    return pl.pallas_call(
        _final_kernel,
        grid=(B, S // 512),
        in_specs=[blk, blk],
        out_specs=blk,
        out_shape=jax.ShapeDtypeStruct((B, S, H), jnp.float32),
    )(hs, mixed)


def kernel(hidden_states, input_injection, cos, sin, Wg, Wd, Wu, Wqkv, Wo, Wgu, Wdn):
    topi, wv, aux = _route(hidden_states[:, 0, :], input_injection[:, 0, :],
                           Wg.T)
    hs = _make_hs(hidden_states, input_injection)
    out = _moe(topi, wv, hs, cos, sin, Wd, Wqkv, Wo, Wgu, Wdn, Wu)
    return out, aux.reshape(())


# speedup vs baseline: 1.0659x; 1.0054x over previous
import dataclasses
import functools

import jax
import jax.numpy as jnp
import numpy as np
from jax.experimental import pallas as pl
from jax.experimental.pallas import tpu as pltpu
from jax.experimental.pallas import tpu_sc as plsc

B, S, H = 2, 2048, 1024
NHS, HD = 4, 64
HS = NHS * HD
E, K = 8, 2
INTER = 768
EPS = 1e-05

CHUNK = 256
NCHUNK = S // CHUNK


def _rms(x):
    var = jnp.mean(x * x, axis=-1, keepdims=True)
    return x * jax.lax.rsqrt(var + EPS)


def _add_kernel(a_ref, b_ref, o_ref):
    o_ref[...] = a_ref[...] + b_ref[...]


def _make_hs(hidden, inj):
    blk = pl.BlockSpec((1, 512, H), lambda b, c: (b, c, 0))
    return pl.pallas_call(
        _add_kernel,
        grid=(B, S // 512),
        in_specs=[blk, blk],
        out_specs=blk,
        out_shape=jax.ShapeDtypeStruct((B, S, H), jnp.float32),
    )(hidden, inj)


def _route(h0, i0, wgT):
    mesh = plsc.VectorSubcoreMesh(core_axis_name="c", subcore_axis_name="s")
    out_type = [
        jax.ShapeDtypeStruct((1, 16), jnp.int32),
        jax.ShapeDtypeStruct((1, 16), jnp.float32),
        jax.ShapeDtypeStruct((1, 16), jnp.float32),
    ]

    cp = pltpu.CompilerParams()
    if "needs_layout_passes" in pltpu.CompilerParams.__dataclass_fields__:
        cp = dataclasses.replace(cp, needs_layout_passes=False)

    @functools.partial(
        pl.kernel, out_type=out_type, mesh=mesh, compiler_params=cp,
        scratch_types=[
            pltpu.VMEM((B, H), jnp.float32),
            pltpu.VMEM((B, H), jnp.float32),
            pltpu.VMEM((E, H), jnp.float32),
            pltpu.VMEM((1, 16), jnp.int32),
            pltpu.VMEM((1, 16), jnp.float32),
            pltpu.VMEM((1, 16), jnp.float32),
            pltpu.SemaphoreType.DMA,
        ])
    def krn(h0_hbm, i0_hbm, wg_hbm, topi_hbm, wv_hbm, aux_hbm,
            h0_v, i0_v, wg_v, ti_v, wv_v, aux_v, sem):
        cid = jax.lax.axis_index("c")
        sid = jax.lax.axis_index("s")

        @pl.when(jnp.logical_and(cid == 0, sid == 0))
        def _():
            pltpu.async_copy(h0_hbm, h0_v, sem).wait()
            pltpu.async_copy(i0_hbm, i0_v, sem).wait()
            pltpu.async_copy(wg_hbm, wg_v, sem).wait()
            lanes = jax.lax.iota(jnp.int32, 16)
            imp = jnp.zeros((16,), jnp.float32)
            ld = jnp.zeros((16,), jnp.float32)
            ti_vec = jnp.zeros((16,), jnp.int32)
            wv_vec = jnp.zeros((16,), jnp.float32)

            @pl.loop(0, H // 64)
            def _(j):
                off = pl.multiple_of(j * 64, 64)
                for u in range(4):
                    sl = pl.ds(off + u * 16, 16)
                    for b_ in range(B):
                        h0_v[b_, sl] = h0_v[b_, sl] + i0_v[b_, sl]

            for b_ in range(B):
                lg = jnp.full((16,), -3e38, jnp.float32)
                for e_ in range(E):
                    def body(j, acc):
                        off = pl.multiple_of(j * 64, 64)
                        for u in range(4):
                            sl = pl.ds(off + u * 16, 16)
                            acc = acc + h0_v[b_, sl] * wg_v[e_, sl]
                        return acc
                    acc = jax.lax.fori_loop(0, H // 64, body,
                                            jnp.zeros((16,), jnp.float32))
                    lg = jnp.where(lanes == e_, jnp.sum(acc), lg)
                m = jnp.max(lg)
                p = jnp.where(lanes < E, jnp.exp(lg - m), 0.0)
                p = p / jnp.broadcast_to(jnp.sum(p), (16,))
                v1 = jnp.max(p)
                i1 = jnp.min(jnp.where(p == v1, lanes, 16))
                p2 = jnp.where(lanes == i1, -1.0, p)
                v2 = jnp.max(p2)
                i2 = jnp.min(jnp.where(p2 == v2, lanes, 16))
                den = jnp.broadcast_to(jnp.maximum(v1 + v2, 1e-08), (16,))
                num = (jnp.where(lanes == K * b_, v1, 0.0)
                       + jnp.where(lanes == K * b_ + 1, v2, 0.0))
                ti_vec = jnp.where(lanes == K * b_, i1, ti_vec)
                ti_vec = jnp.where(lanes == K * b_ + 1, i2, ti_vec)
                wv_vec = wv_vec + num / den
                imp = imp + p
                ld = ld + jnp.where(
                    jnp.logical_or(lanes == i1, lanes == i2), 1.0, 0.0)
            aux = jnp.sum(imp * ld) * (float(E) / (B * B * K))
            ti_v[0, pl.ds(0, 16)] = ti_vec
            wv_v[0, pl.ds(0, 16)] = wv_vec
            aux_v[0, pl.ds(0, 16)] = jnp.where(lanes == 0, aux, 0.0)
            pltpu.async_copy(ti_v, topi_hbm, sem).wait()
            pltpu.async_copy(wv_v, wv_hbm, sem).wait()
            pltpu.async_copy(aux_v, aux_hbm, sem).wait()

    ti_raw, wv_raw, aux_raw = krn(h0, i0, wgT)
    topi = ti_raw[0, :B * K].reshape(B, K)
    wv = wv_raw[0, :B * K].reshape(B, K)
    return topi, wv, aux_raw[0:1, 0:1]


def _moe_kernel(topi_ref, wv_ref,
                hs_ref, cos_ref, sin_ref,
                wd_ref, wqkv_ref, wo_ref, wgu_ref, wdn_ref, wu_ref,
                out_ref,
                h_s, q_s, k_s, v_s):
    b = pl.program_id(0)
    kk = pl.program_id(1)
    wgt = wv_ref[b, kk]

    bf = jnp.bfloat16
    wd = wd_ref[0].astype(bf)
    wqkv = wqkv_ref[0].astype(bf)
    scale = np.float32(1.0 / np.sqrt(HD))

    def rope_head(xh, cosc, sinc):
        x1 = xh[:, :HD // 2]
        x2 = xh[:, HD // 2:]
        rot = jnp.concatenate([-x2, x1], axis=1)
        return xh * cosc + rot * sinc

    def phase1(c, carry):
        rows = pl.ds(c * CHUNK, CHUNK)
        x = hs_ref[0, rows, :].astype(bf)
        hc = jnp.dot(x, wd, preferred_element_type=jnp.float32)
        h_s[rows, :] = hc
        qkv = jnp.dot(hc.astype(bf), wqkv,
                      preferred_element_type=jnp.float32)
        cosc = cos_ref[rows, :]
        sinc = sin_ref[rows, :]
        for hh in range(NHS):
            qh = qkv[:, hh * HD:(hh + 1) * HD]
            kh = qkv[:, HS + hh * HD:HS + (hh + 1) * HD]
            vh = qkv[:, 2 * HS + hh * HD:2 * HS + (hh + 1) * HD]
            q_s[hh, rows, :] = (rope_head(qh, cosc, sinc) * scale).astype(bf)
            k_s[hh, rows, :] = rope_head(kh, cosc, sinc).astype(bf)
            v_s[hh, rows, :] = vh.astype(bf)
        return carry

    jax.lax.fori_loop(0, NCHUNK, phase1, 0)

    wo = wo_ref[0].astype(bf)
    wgu = wgu_ref[0].astype(bf)
    wdn = wdn_ref[0].astype(bf)
    wu = wu_ref[0].astype(bf)

    def phase2(c, carry):
        rows = pl.ds(c * CHUNK, CHUNK)
        hc = h_s[rows, :]
        o_heads = []
        for hh in range(NHS):
            qh = q_s[hh, rows, :]
            kh = k_s[hh]
            vh = v_s[hh]
            scores = jax.lax.dot_general(
                qh, kh, (((1,), (1,)), ((), ())),
                preferred_element_type=jnp.float32)
            mx = jnp.max(scores, axis=1, keepdims=True)
            ee = jnp.exp(scores - mx)
            rs = 1.0 / jnp.sum(ee, axis=1, keepdims=True)
            ov = jnp.dot(ee.astype(bf), vh,
                         preferred_element_type=jnp.float32)
            o_heads.append(ov * rs)
        o = jnp.concatenate(o_heads, axis=1).astype(bf)
        o = jnp.dot(o, wo, preferred_element_type=jnp.float32)
        t = _rms(hc + o)
        gu = jnp.dot(t.astype(bf), wgu,
                     preferred_element_type=jnp.float32)
        g = gu[:, :INTER]
        u = gu[:, INTER:]
        mm = jnp.dot((jax.nn.silu(g) * u).astype(bf), wdn,
                     preferred_element_type=jnp.float32)
        t2 = _rms(t + mm)
        oe = jnp.dot(t2.astype(bf), wu, preferred_element_type=jnp.float32)

        @pl.when(kk == 0)
        def _():
            out_ref[0, rows, :] = wgt * oe

        @pl.when(kk == K - 1)
        def _():
            x = hs_ref[0, rows, :] + out_ref[0, rows, :] + wgt * oe
            var = jnp.mean(x * x, axis=-1, keepdims=True)
            out_ref[0, rows, :] = x * jax.lax.rsqrt(var + EPS)

        return carry

    jax.lax.fori_loop(0, NCHUNK, phase2, 0)


def _moe(topi, wv, hs, cos, sin, Wd, Wqkv, Wo, Wgu, Wdn, Wu):
    grid_spec = pltpu.PrefetchScalarGridSpec(
        num_scalar_prefetch=2,
        grid=(B, K),
        in_specs=[
            pl.BlockSpec((1, S, H), lambda b, k, ti, wv_: (b, 0, 0)),
            pl.BlockSpec((S, HD), lambda b, k, ti, wv_: (0, 0)),
            pl.BlockSpec((S, HD), lambda b, k, ti, wv_: (0, 0)),
            pl.BlockSpec((1, H, HS), lambda b, k, ti, wv_: (ti[b, k], 0, 0)),
            pl.BlockSpec((1, HS, 3 * HS), lambda b, k, ti, wv_: (ti[b, k], 0, 0)),
            pl.BlockSpec((1, HS, HS), lambda b, k, ti, wv_: (ti[b, k], 0, 0)),
            pl.BlockSpec((1, HS, 2 * INTER), lambda b, k, ti, wv_: (ti[b, k], 0, 0)),
            pl.BlockSpec((1, INTER, HS), lambda b, k, ti, wv_: (ti[b, k], 0, 0)),
            pl.BlockSpec((1, HS, H), lambda b, k, ti, wv_: (ti[b, k], 0, 0)),
        ],
        out_specs=pl.BlockSpec((1, S, H), lambda b, k, ti, wv_: (b, 0, 0)),
        scratch_shapes=[
            pltpu.VMEM((S, HS), jnp.float32),
            pltpu.VMEM((NHS, S, HD), jnp.bfloat16),
            pltpu.VMEM((NHS, S, HD), jnp.bfloat16),
            pltpu.VMEM((NHS, S, HD), jnp.bfloat16),
        ],
    )
    return pl.pallas_call(
        _moe_kernel,
        grid_spec=grid_spec,
        out_shape=jax.ShapeDtypeStruct((B, S, H), jnp.float32),
        compiler_params=pltpu.CompilerParams(
            vmem_limit_bytes=64 * 1024 * 1024,
            dimension_semantics=("parallel", "arbitrary"),
        ),
    )(topi, wv, hs, cos, sin, Wd, Wqkv, Wo, Wgu, Wdn, Wu)


def _final_kernel(hs_ref, mix_ref, o_ref):
    x = hs_ref[...] + mix_ref[...]
    var = jnp.mean(x * x, axis=-1, keepdims=True)
    o_ref[...] = x * jax.lax.rsqrt(var + EPS)


def _finalize(hs, mixed):
    blk = pl.BlockSpec((1, 512, H), lambda b, c: (b, c, 0))
    return pl.pallas_call(
        _final_kernel,
        grid=(B, S // 512),
        in_specs=[blk, blk],
        out_specs=blk,
        out_shape=jax.ShapeDtypeStruct((B, S, H), jnp.float32),
    )(hs, mixed)


def kernel(hidden_states, input_injection, cos, sin, Wg, Wd, Wu, Wqkv, Wo, Wgu, Wdn):
    topi, wv, aux = _route(hidden_states[:, 0, :], input_injection[:, 0, :],
                           Wg.T)
    hs = _make_hs(hidden_states, input_injection)
    out = _moe(topi, wv, hs, cos, sin, Wd, Wqkv, Wo, Wgu, Wdn, Wu)
    return out, aux.reshape(())
